# Initial kernel scaffold; baseline (speedup 1.0000x reference)
import jax
import jax.numpy as jnp
from jax.experimental import pallas as pl


def kernel(features, edge_index, W, b):
    raise NotImplementedError("write your pallas kernel here")


# speedup vs baseline: 4.1994x; 4.1994x over previous
import jax
import jax.numpy as jnp
from jax import lax
from jax.experimental import pallas as pl
from jax.experimental.pallas import tpu as pltpu
from jax.experimental.pallas import tpu_sc as plsc

N_NODES = 10000
N_EDGES = 320000
D = 128

NC = 2
NS = 16
NW = NC * NS
E_PER_TILE = N_EDGES // NW
CHUNK = 128
N_CHUNKS = 80
E_PAD = N_CHUNKS * CHUNK
NJ = N_CHUNKS // 2
N_PAD = 10240
ROWS_PER_TILE = N_PAD // NS


def _sc_body(feat_hbm, idx_hbm, zeros_hbm, out_hbm,
             idx0, idx1, rows0, rows1, acc, si0, si1, sg0, sg1):
    cid = lax.axis_index("c")
    sid = lax.axis_index("s")
    wid = cid * NS + sid
    row_base = sid * ROWS_PER_TILE

    pltpu.async_copy(idx_hbm.at[wid, 0], idx0, si0)
    pltpu.async_copy(idx_hbm.at[wid, 1], idx1, si1)
    pltpu.sync_copy(zeros_hbm, acc.at[pl.ds(row_base, ROWS_PER_TILE)])
    plsc.subcore_barrier()

    pltpu.make_async_copy(idx_hbm.at[wid, 0], idx0, si0).wait()
    pltpu.async_copy(feat_hbm.at[idx0.at[0]], rows0, sg0)

    def step(j, carry):
        c0 = 2 * j
        c1 = c0 + 1
        pltpu.make_async_copy(idx_hbm.at[wid, c1], idx1, si1).wait()
        pltpu.async_copy(feat_hbm.at[idx1.at[0]], rows1, sg1)
        pltpu.make_async_copy(feat_hbm.at[idx0.at[0]], rows0, sg0).wait()
        pltpu.sync_copy(rows0, acc.at[idx0.at[1]], add=True)

        @pl.when(j < NJ - 1)
        def _():
            pltpu.async_copy(idx_hbm.at[wid, c0 + 2], idx0, si0)
            pltpu.make_async_copy(idx_hbm.at[wid, c0 + 2], idx0, si0).wait()
            pltpu.async_copy(feat_hbm.at[idx0.at[0]], rows0, sg0)

        pltpu.make_async_copy(feat_hbm.at[idx1.at[0]], rows1, sg1).wait()
        pltpu.sync_copy(rows1, acc.at[idx1.at[1]], add=True)

        @pl.when(j < NJ - 1)
        def _():
            pltpu.async_copy(idx_hbm.at[wid, c1 + 2], idx1, si1)

        return carry

    lax.fori_loop(0, NJ, step, 0)

    plsc.subcore_barrier()
    pltpu.sync_copy(acc.at[pl.ds(row_base, ROWS_PER_TILE)],
                    out_hbm.at[cid, pl.ds(row_base, ROWS_PER_TILE)])


def _sc_aggregate(features, idx):
    mesh = plsc.VectorSubcoreMesh(core_axis_name="c", subcore_axis_name="s")
    zeros = jnp.zeros((ROWS_PER_TILE, D), jnp.float32)
    return pl.kernel(
        _sc_body,
        out_type=jax.ShapeDtypeStruct((NC, N_PAD, D), jnp.float32),
        mesh=mesh,
        scratch_types=[
            pltpu.VMEM((2, CHUNK), jnp.int32),
            pltpu.VMEM((2, CHUNK), jnp.int32),
            pltpu.VMEM((CHUNK, D), jnp.float32),
            pltpu.VMEM((CHUNK, D), jnp.float32),
            pltpu.VMEM_SHARED((N_PAD, D), jnp.float32),
            pltpu.SemaphoreType.DMA,
            pltpu.SemaphoreType.DMA,
            pltpu.SemaphoreType.DMA,
            pltpu.SemaphoreType.DMA,
        ],
    )(features, idx, zeros)


ROW_BLK = 1000


def _tc_body(p_ref, w_ref, b_ref, o_ref):
    agg = p_ref[0] + p_ref[1]
    o_ref[...] = (
        jnp.dot(agg, w_ref[...], preferred_element_type=jnp.float32)
        + b_ref[...]
    )


def _tc_linear(partials, W, b):
    return pl.pallas_call(
        _tc_body,
        grid=(N_NODES // ROW_BLK,),
        in_specs=[
            pl.BlockSpec((NC, ROW_BLK, D), lambda i: (0, i, 0)),
            pl.BlockSpec((D, D), lambda i: (0, 0)),
            pl.BlockSpec((1, D), lambda i: (0, 0)),
        ],
        out_specs=pl.BlockSpec((ROW_BLK, D), lambda i: (i, 0)),
        out_shape=jax.ShapeDtypeStruct((N_NODES, D), jnp.float32),
    )(partials, W, b.reshape(1, D))


def kernel(features, edge_index, W, b):
    src = edge_index[0].astype(jnp.int32).reshape(NW, E_PER_TILE)
    dst = edge_index[1].astype(jnp.int32).reshape(NW, E_PER_TILE)
    pad = E_PAD - E_PER_TILE
    src3 = jnp.pad(src, ((0, 0), (0, pad))).reshape(NW, N_CHUNKS, CHUNK)
    dst3 = jnp.pad(dst, ((0, 0), (0, pad)),
                   constant_values=N_NODES).reshape(NW, N_CHUNKS, CHUNK)
    idx = jnp.stack([src3, dst3], axis=2)
    partials = _sc_aggregate(features, idx)
    return _tc_linear(partials, W, b)
